# baseline (speedup 1.0000x reference)
import functools

import jax
import jax.numpy as jnp
from jax import lax
from jax.experimental import pallas as pl
from jax.experimental.pallas import tpu as pltpu
from jax.experimental.pallas import tpu_sc as plsc

N = 10000
D = 128
NP = 10112
NC = 2
NS = 16
NW = NC * NS
ROWS_PT = 80
EROWS = NW * ROWS_PT
EROWS_REAL = 320000 // 128
RPS = NP // NS
RB = 2000

_mesh = plsc.VectorSubcoreMesh(
    core_axis_name="c", subcore_axis_name="s", num_cores=NC, num_subcores=NS)


def _wid():
    return lax.axis_index("s") * NC + lax.axis_index("c")


def _real_rows(wid):
    return jnp.clip(EROWS_REAL - wid * ROWS_PT, 0, ROWS_PT)


def _sc_count_body(ei_hbm, hist_hbm, didx_v, hist_v, hist2_v):
    wid = _wid()

    def zero(i, _):
        hist_v[pl.ds(i * 16, 16)] = jnp.zeros((16,), jnp.float32)
        hist2_v[pl.ds(i * 16, 16)] = jnp.zeros((16,), jnp.float32)
        return 0
    lax.fori_loop(0, NP // 16, zero, 0)

    pltpu.sync_copy(ei_hbm.at[1, pl.ds(wid * ROWS_PT, ROWS_PT)], didx_v)
    ones = jnp.ones((16,), jnp.float32)

    def row(r, _):
        for j in range(0, 8, 2):
            plsc.addupdate_scatter(
                hist_v, [didx_v[r, pl.ds(j * 16, 16)]], ones)
            plsc.addupdate_scatter(
                hist2_v, [didx_v[r, pl.ds(j * 16 + 16, 16)]], ones)
        return 0
    lax.fori_loop(0, _real_rows(wid), row, 0)

    def merge(i, _):
        s = pl.ds(i * 16, 16)
        hist_v[s] = hist_v[s] + hist2_v[s]
        return 0
    lax.fori_loop(0, NP // 16, merge, 0)

    pltpu.sync_copy(hist_v, hist_hbm.at[wid])


_sc_count = functools.partial(
    pl.kernel,
    out_type=jax.ShapeDtypeStruct((NW, NP), jnp.float32),
    mesh=_mesh,
    compiler_params=pltpu.CompilerParams(needs_layout_passes=False),
    scratch_types=[
        pltpu.VMEM((ROWS_PT, 128), jnp.int32),
        pltpu.VMEM((NP,), jnp.float32),
        pltpu.VMEM((NP,), jnp.float32),
    ],
)(_sc_count_body)


def _sc_agg_body(y_hbm, ei_hbm, acc0_hbm, acc1_hbm,
                 sidx_v, didx_v, rows_v, acc_sh, gsem, ssem):
    cid = lax.axis_index("c")
    sid = lax.axis_index("s")
    wid = _wid()

    base = sid * RPS

    @pl.when(cid == 0)
    def _():
        @pl.when(base + RPS <= N)
        def _():
            pltpu.sync_copy(y_hbm.at[pl.ds(base, RPS)],
                            acc_sh.at[pl.ds(base, RPS)])

        @pl.when(base + RPS > N)
        def _():
            pltpu.sync_copy(y_hbm.at[pl.ds(N - (RPS - 112), RPS - 112)],
                            acc_sh.at[pl.ds(N - (RPS - 112), RPS - 112)])

    @pl.when(cid == 1)
    def _():
        def zrow(i, _):
            for j in range(8):
                rows_v[0, i, pl.ds(j * 16, 16)] = jnp.zeros((16,), jnp.float32)
            return 0
        lax.fori_loop(0, 128, zrow, 0)
        for k in range(4):
            pltpu.sync_copy(rows_v.at[0],
                            acc_sh.at[pl.ds(base + k * 128, 128)])
        pltpu.sync_copy(rows_v.at[0, pl.ds(0, RPS - 512)],
                        acc_sh.at[pl.ds(base + 512, RPS - 512)])
    plsc.subcore_barrier()

    pltpu.sync_copy(ei_hbm.at[0, pl.ds(wid * ROWS_PT, ROWS_PT)], sidx_v)

    rw = _real_rows(wid)

    @pl.when(rw > 0)
    def _():
        pltpu.async_copy(y_hbm.at[sidx_v.at[0]], rows_v.at[0], gsem)

    def group(g, _):
        gn = jnp.minimum(8, rw - g * 8)
        gb = lax.rem(g, 2)
        pltpu.sync_copy(
            ei_hbm.at[1, pl.ds(wid * ROWS_PT + g * 8, 8)], didx_v.at[gb])

        def chunk(j, _):
            c = g * 8 + j
            buf = lax.rem(c, 2)
            pltpu.make_async_copy(
                y_hbm.at[sidx_v.at[c]], rows_v.at[buf], gsem).wait()

            @pl.when(c >= 1)
            def _():
                pltpu.make_async_copy(
                    rows_v.at[1 - buf], acc_sh.at[didx_v.at[gb, j]],
                    ssem).wait()

            @pl.when(c + 1 < rw)
            def _():
                pltpu.async_copy(
                    y_hbm.at[sidx_v.at[c + 1]], rows_v.at[1 - buf], gsem)
            pltpu.async_copy(
                rows_v.at[buf], acc_sh.at[didx_v.at[gb, j]], ssem, add=True)
            return 0
        lax.fori_loop(0, gn, chunk, 0)
        return 0
    lax.fori_loop(0, (rw + 7) // 8, group, 0)

    @pl.when(rw > 0)
    def _():
        pltpu.make_async_copy(
            rows_v.at[0], acc_sh.at[didx_v.at[0, 0]], ssem).wait()

    plsc.subcore_barrier()

    @pl.when(cid == 0)
    def _():
        pltpu.sync_copy(acc_sh.at[pl.ds(base, RPS)],
                        acc0_hbm.at[pl.ds(base, RPS)])

    @pl.when(cid == 1)
    def _():
        pltpu.sync_copy(acc_sh.at[pl.ds(base, RPS)],
                        acc1_hbm.at[pl.ds(base, RPS)])


_sc_agg = functools.partial(
    pl.kernel,
    out_type=[jax.ShapeDtypeStruct((NP, D), jnp.float32),
              jax.ShapeDtypeStruct((NP, D), jnp.float32)],
    mesh=_mesh,
    compiler_params=pltpu.CompilerParams(needs_layout_passes=False),
    scratch_types=[
        pltpu.VMEM((ROWS_PT, 128), jnp.int32),
        pltpu.VMEM((2, 8, 128), jnp.int32),
        pltpu.VMEM((2, 128, D), jnp.float32),
        pltpu.VMEM_SHARED((NP, D), jnp.float32),
        pltpu.SemaphoreType.DMA,
        pltpu.SemaphoreType.DMA,
    ],
)(_sc_agg_body)


def _tc_y_body(x_ref, w_ref, h_ref, y_ref):
    xw = jnp.dot(x_ref[...], w_ref[...], preferred_element_type=jnp.float32)
    deg = 1.0 + jnp.sum(h_ref[0], axis=0)
    dinv = lax.rsqrt(deg)
    y_ref[...] = xw * dinv[:, None]


def _tc_y(x, w, hists):
    return pl.pallas_call(
        _tc_y_body,
        grid=(N // RB,),
        in_specs=[
            pl.BlockSpec((RB, D), lambda i: (i, 0)),
            pl.BlockSpec((D, D), lambda i: (0, 0)),
            pl.BlockSpec((1, NW, RB), lambda i: (i, 0, 0)),
        ],
        out_specs=pl.BlockSpec((RB, D), lambda i: (i, 0)),
        out_shape=jax.ShapeDtypeStruct((N, D), jnp.float32),
    )(x, w, hists)


def _tc_head_body(a0, a1, h, x_ref, bc, w1, b1, w2, b2, w3, b3, o_ref):
    deg = 1.0 + jnp.sum(h[0], axis=0)
    dinv = lax.rsqrt(deg)
    g = (a0[...] + a1[...]) * dinv[:, None] + bc[...]
    g = jnp.maximum(g, 0.0) + x_ref[...]
    z = jnp.maximum(
        jnp.dot(g, w1[...], preferred_element_type=jnp.float32) + b1[...], 0.0)
    z = jnp.maximum(
        jnp.dot(z, w2[...], preferred_element_type=jnp.float32) + b2[...], 0.0)
    o_ref[...] = jnp.dot(z, w3[...], preferred_element_type=jnp.float32) + b3[...]


def _tc_head(acc0, acc1, hists, x, bc, w1, b1, w2, b2, w3, b3):
    full = lambda r, c: pl.BlockSpec((r, c), lambda i: (0, 0))
    return pl.pallas_call(
        _tc_head_body,
        grid=(N // RB,),
        in_specs=[
            pl.BlockSpec((RB, D), lambda i: (i, 0)),
            pl.BlockSpec((RB, D), lambda i: (i, 0)),
            pl.BlockSpec((1, NW, RB), lambda i: (i, 0, 0)),
            pl.BlockSpec((RB, D), lambda i: (i, 0)),
            full(1, D), full(D, 32), full(1, 32),
            full(32, 32), full(1, 32), full(32, 2), full(1, 2),
        ],
        out_specs=pl.BlockSpec((RB, 2), lambda i: (i, 0)),
        out_shape=jax.ShapeDtypeStruct((N, 2), jnp.float32),
    )(acc0, acc1, hists, x, bc, w1, b1, w2, b2, w3, b3)


def kernel(x, edge_index, W_conv, b_conv, W1, b1, W2, b2, W3, b3):
    ei = edge_index.astype(jnp.int32)
    erows = ei.shape[1] // 128
    ei3 = jnp.pad(ei.reshape(2, erows, 128),
                  ((0, 0), (0, EROWS - erows), (0, 0)))

    hists = _sc_count(ei3)
    hs3 = hists[:, :N].reshape(NW, N // RB, RB).transpose(1, 0, 2)
    y = _tc_y(x, W_conv, hs3)
    acc0, acc1 = _sc_agg(y, ei3)

    return _tc_head(acc0, acc1, hs3, x, b_conv.reshape(1, D),
                    W1, b1.reshape(1, 32), W2, b2.reshape(1, 32),
                    W3, b3.reshape(1, 2))


# speedup vs baseline: 1.0128x; 1.0128x over previous
import functools

import jax
import jax.numpy as jnp
from jax import lax
from jax.experimental import pallas as pl
from jax.experimental.pallas import tpu as pltpu
from jax.experimental.pallas import tpu_sc as plsc

N = 10000
D = 128
NP = 10112
NC = 2
NS = 16
NW = NC * NS
ROWS_PT = 80
EROWS = NW * ROWS_PT
EROWS_REAL = 320000 // 128
RPS = NP // NS
RB = 2000

_mesh = plsc.VectorSubcoreMesh(
    core_axis_name="c", subcore_axis_name="s", num_cores=NC, num_subcores=NS)


def _wid():
    return lax.axis_index("s") * NC + lax.axis_index("c")


def _real_rows(wid):
    return jnp.clip(EROWS_REAL - wid * ROWS_PT, 0, ROWS_PT)


def _sc_count_body(ei_hbm, hist_hbm, didx_v, hist_v):
    wid = _wid()

    def zero(i, _):
        hist_v[pl.ds(i * 16, 16)] = jnp.zeros((16,), jnp.float32)
        return 0
    lax.fori_loop(0, NP // 16, zero, 0)

    pltpu.sync_copy(ei_hbm.at[1, pl.ds(wid * ROWS_PT, ROWS_PT)], didx_v)
    ones = jnp.ones((16,), jnp.float32)

    def row(r, _):
        for j in range(8):
            idx = didx_v[r, pl.ds(j * 16, 16)]
            plsc.addupdate_scatter(hist_v, [idx], ones)
        return 0
    lax.fori_loop(0, _real_rows(wid), row, 0)

    pltpu.sync_copy(hist_v, hist_hbm.at[wid])


_sc_count = functools.partial(
    pl.kernel,
    out_type=jax.ShapeDtypeStruct((NW, NP), jnp.float32),
    mesh=_mesh,
    compiler_params=pltpu.CompilerParams(needs_layout_passes=False),
    scratch_types=[
        pltpu.VMEM((ROWS_PT, 128), jnp.int32),
        pltpu.VMEM((NP,), jnp.float32),
    ],
)(_sc_count_body)


def _sc_agg_body(y_hbm, ei_hbm, acc0_hbm, acc1_hbm,
                 sidx_v, didx_v, rows_v, acc_sh, gsem, ssem):
    cid = lax.axis_index("c")
    sid = lax.axis_index("s")
    wid = _wid()

    base = sid * RPS

    @pl.when(cid == 0)
    def _():
        @pl.when(base + RPS <= N)
        def _():
            pltpu.sync_copy(y_hbm.at[pl.ds(base, RPS)],
                            acc_sh.at[pl.ds(base, RPS)])

        @pl.when(base + RPS > N)
        def _():
            pltpu.sync_copy(y_hbm.at[pl.ds(N - (RPS - 112), RPS - 112)],
                            acc_sh.at[pl.ds(N - (RPS - 112), RPS - 112)])

    @pl.when(cid == 1)
    def _():
        def zrow(i, _):
            for j in range(8):
                rows_v[0, i, pl.ds(j * 16, 16)] = jnp.zeros((16,), jnp.float32)
            return 0
        lax.fori_loop(0, 128, zrow, 0)
        for k in range(4):
            pltpu.sync_copy(rows_v.at[0],
                            acc_sh.at[pl.ds(base + k * 128, 128)])
        pltpu.sync_copy(rows_v.at[0, pl.ds(0, RPS - 512)],
                        acc_sh.at[pl.ds(base + 512, RPS - 512)])
    plsc.subcore_barrier()

    pltpu.sync_copy(ei_hbm.at[0, pl.ds(wid * ROWS_PT, ROWS_PT)], sidx_v)

    rw = _real_rows(wid)

    @pl.when(rw > 0)
    def _():
        pltpu.async_copy(y_hbm.at[sidx_v.at[0]], rows_v.at[0], gsem)

    def group(g, _):
        gn = jnp.minimum(8, rw - g * 8)
        gb = lax.rem(g, 2)
        pltpu.sync_copy(
            ei_hbm.at[1, pl.ds(wid * ROWS_PT + g * 8, 8)], didx_v.at[gb])

        def chunk(j, _):
            c = g * 8 + j
            buf = lax.rem(c, 2)
            pltpu.make_async_copy(
                y_hbm.at[sidx_v.at[c]], rows_v.at[buf], gsem).wait()

            @pl.when(c >= 1)
            def _():
                pltpu.make_async_copy(
                    rows_v.at[1 - buf], acc_sh.at[didx_v.at[gb, j]],
                    ssem).wait()

            @pl.when(c + 1 < rw)
            def _():
                pltpu.async_copy(
                    y_hbm.at[sidx_v.at[c + 1]], rows_v.at[1 - buf], gsem)
            pltpu.async_copy(
                rows_v.at[buf], acc_sh.at[didx_v.at[gb, j]], ssem, add=True)
            return 0
        lax.fori_loop(0, gn, chunk, 0)
        return 0
    lax.fori_loop(0, (rw + 7) // 8, group, 0)

    @pl.when(rw > 0)
    def _():
        pltpu.make_async_copy(
            rows_v.at[0], acc_sh.at[didx_v.at[0, 0]], ssem).wait()

    plsc.subcore_barrier()

    @pl.when(cid == 0)
    def _():
        pltpu.sync_copy(acc_sh.at[pl.ds(base, RPS)],
                        acc0_hbm.at[pl.ds(base, RPS)])

    @pl.when(cid == 1)
    def _():
        pltpu.sync_copy(acc_sh.at[pl.ds(base, RPS)],
                        acc1_hbm.at[pl.ds(base, RPS)])


_sc_agg = functools.partial(
    pl.kernel,
    out_type=[jax.ShapeDtypeStruct((NP, D), jnp.float32),
              jax.ShapeDtypeStruct((NP, D), jnp.float32)],
    mesh=_mesh,
    compiler_params=pltpu.CompilerParams(needs_layout_passes=False),
    scratch_types=[
        pltpu.VMEM((ROWS_PT, 128), jnp.int32),
        pltpu.VMEM((2, 8, 128), jnp.int32),
        pltpu.VMEM((2, 128, D), jnp.float32),
        pltpu.VMEM_SHARED((NP, D), jnp.float32),
        pltpu.SemaphoreType.DMA,
        pltpu.SemaphoreType.DMA,
    ],
)(_sc_agg_body)


def _tc_y_body(x_ref, w_ref, h_ref, y_ref):
    xw = jnp.dot(x_ref[...], w_ref[...], preferred_element_type=jnp.float32)
    deg = 1.0 + jnp.sum(h_ref[0], axis=0)
    dinv = lax.rsqrt(deg)
    y_ref[...] = xw * dinv[:, None]


def _tc_y(x, w, hists):
    return pl.pallas_call(
        _tc_y_body,
        grid=(N // RB,),
        in_specs=[
            pl.BlockSpec((RB, D), lambda i: (i, 0)),
            pl.BlockSpec((D, D), lambda i: (0, 0)),
            pl.BlockSpec((1, NW, RB), lambda i: (i, 0, 0)),
        ],
        out_specs=pl.BlockSpec((RB, D), lambda i: (i, 0)),
        out_shape=jax.ShapeDtypeStruct((N, D), jnp.float32),
    )(x, w, hists)


def _tc_head_body(a0, a1, h, x_ref, bc, w1, b1, w2, b2, w3, b3, o_ref):
    deg = 1.0 + jnp.sum(h[0], axis=0)
    dinv = lax.rsqrt(deg)
    g = (a0[...] + a1[...]) * dinv[:, None] + bc[...]
    g = jnp.maximum(g, 0.0) + x_ref[...]
    z = jnp.maximum(
        jnp.dot(g, w1[...], preferred_element_type=jnp.float32) + b1[...], 0.0)
    z = jnp.maximum(
        jnp.dot(z, w2[...], preferred_element_type=jnp.float32) + b2[...], 0.0)
    o_ref[...] = jnp.dot(z, w3[...], preferred_element_type=jnp.float32) + b3[...]


def _tc_head(acc0, acc1, hists, x, bc, w1, b1, w2, b2, w3, b3):
    full = lambda r, c: pl.BlockSpec((r, c), lambda i: (0, 0))
    return pl.pallas_call(
        _tc_head_body,
        grid=(N // RB,),
        in_specs=[
            pl.BlockSpec((RB, D), lambda i: (i, 0)),
            pl.BlockSpec((RB, D), lambda i: (i, 0)),
            pl.BlockSpec((1, NW, RB), lambda i: (i, 0, 0)),
            pl.BlockSpec((RB, D), lambda i: (i, 0)),
            full(1, D), full(D, 32), full(1, 32),
            full(32, 32), full(1, 32), full(32, 2), full(1, 2),
        ],
        out_specs=pl.BlockSpec((RB, 2), lambda i: (i, 0)),
        out_shape=jax.ShapeDtypeStruct((N, 2), jnp.float32),
    )(acc0, acc1, hists, x, bc, w1, b1, w2, b2, w3, b3)


def kernel(x, edge_index, W_conv, b_conv, W1, b1, W2, b2, W3, b3):
    ei = edge_index.astype(jnp.int32)
    erows = ei.shape[1] // 128
    ei3 = jnp.pad(ei.reshape(2, erows, 128),
                  ((0, 0), (0, EROWS - erows), (0, 0)))

    hists = _sc_count(ei3)
    hs3 = hists[:, :N].reshape(NW, N // RB, RB).transpose(1, 0, 2)
    y = _tc_y(x, W_conv, hs3)
    acc0, acc1 = _sc_agg(y, ei3)

    return _tc_head(acc0, acc1, hs3, x, b_conv.reshape(1, D),
                    W1, b1.reshape(1, 32), W2, b2.reshape(1, 32),
                    W3, b3.reshape(1, 2))


# speedup vs baseline: 1.0248x; 1.0119x over previous
import functools

import jax
import jax.numpy as jnp
from jax import lax
from jax.experimental import pallas as pl
from jax.experimental.pallas import tpu as pltpu
from jax.experimental.pallas import tpu_sc as plsc

N = 10000
D = 128
NP = 10112
NC = 2
NS = 16
NW = NC * NS
ROWS_PT = 80
EROWS = NW * ROWS_PT
EROWS_REAL = 320000 // 128
RPS = NP // NS
RB = 2000

_mesh = plsc.VectorSubcoreMesh(
    core_axis_name="c", subcore_axis_name="s", num_cores=NC, num_subcores=NS)


def _wid():
    return lax.axis_index("s") * NC + lax.axis_index("c")


def _real_rows(wid):
    return jnp.clip(EROWS_REAL - wid * ROWS_PT, 0, ROWS_PT)


def _sc_count_body(ei_hbm, hist_hbm, didx_v, hist_v):
    wid = _wid()

    def zero(i, _):
        hist_v[pl.ds(i * 16, 16)] = jnp.zeros((16,), jnp.float32)
        return 0
    lax.fori_loop(0, NP // 16, zero, 0)

    pltpu.sync_copy(ei_hbm.at[1, pl.ds(wid * ROWS_PT, ROWS_PT)], didx_v)
    ones = jnp.ones((16,), jnp.float32)

    def row(r, _):
        for j in range(8):
            idx = didx_v[r, pl.ds(j * 16, 16)]
            plsc.addupdate_scatter(hist_v, [idx], ones)
        return 0
    lax.fori_loop(0, _real_rows(wid), row, 0)

    pltpu.sync_copy(hist_v, hist_hbm.at[wid])


_sc_count = functools.partial(
    pl.kernel,
    out_type=jax.ShapeDtypeStruct((NW, NP), jnp.float32),
    mesh=_mesh,
    compiler_params=pltpu.CompilerParams(needs_layout_passes=False),
    scratch_types=[
        pltpu.VMEM((ROWS_PT, 128), jnp.int32),
        pltpu.VMEM((NP,), jnp.float32),
    ],
)(_sc_count_body)


def _sc_agg_body(y_hbm, ei_hbm, acc0_hbm, acc1_hbm,
                 sidx_v, didx_v, rows_v, acc_sh, gsem, ssem):
    cid = lax.axis_index("c")
    sid = lax.axis_index("s")
    wid = _wid()

    base = sid * RPS

    def zrow(i, _):
        for j in range(8):
            rows_v[0, i, pl.ds(j * 16, 16)] = jnp.zeros((16,), jnp.float32)
        return 0
    lax.fori_loop(0, 128, zrow, 0)

    for k in range(4):
        pltpu.sync_copy(rows_v.at[0], acc_sh.at[pl.ds(base + k * 128, 128)])
    pltpu.sync_copy(rows_v.at[0, pl.ds(0, RPS - 512)],
                    acc_sh.at[pl.ds(base + 512, RPS - 512)])
    plsc.subcore_barrier()

    pltpu.sync_copy(ei_hbm.at[0, pl.ds(wid * ROWS_PT, ROWS_PT)], sidx_v)

    rw = _real_rows(wid)

    @pl.when(rw > 0)
    def _():
        pltpu.async_copy(y_hbm.at[sidx_v.at[0]], rows_v.at[0], gsem)

    def group(g, _):
        gn = jnp.minimum(8, rw - g * 8)
        pltpu.sync_copy(
            ei_hbm.at[1, pl.ds(wid * ROWS_PT + g * 8, 8)], didx_v.at[0])

        def chunk(j, _):
            c = g * 8 + j
            buf = lax.rem(c, 2)
            pltpu.make_async_copy(
                y_hbm.at[sidx_v.at[c]], rows_v.at[buf], gsem).wait()

            @pl.when(c + 1 < rw)
            def _():
                pltpu.async_copy(
                    y_hbm.at[sidx_v.at[c + 1]], rows_v.at[1 - buf], gsem)
            pltpu.sync_copy(rows_v.at[buf], acc_sh.at[didx_v.at[0, j]],
                            add=True)
            return 0
        lax.fori_loop(0, gn, chunk, 0)
        return 0
    lax.fori_loop(0, (rw + 7) // 8, group, 0)

    plsc.subcore_barrier()

    @pl.when(cid == 0)
    def _():
        pltpu.sync_copy(acc_sh.at[pl.ds(base, RPS)],
                        acc0_hbm.at[pl.ds(base, RPS)])

    @pl.when(cid == 1)
    def _():
        pltpu.sync_copy(acc_sh.at[pl.ds(base, RPS)],
                        acc1_hbm.at[pl.ds(base, RPS)])


_sc_agg = functools.partial(
    pl.kernel,
    out_type=[jax.ShapeDtypeStruct((NP, D), jnp.float32),
              jax.ShapeDtypeStruct((NP, D), jnp.float32)],
    mesh=_mesh,
    compiler_params=pltpu.CompilerParams(needs_layout_passes=False),
    scratch_types=[
        pltpu.VMEM((ROWS_PT, 128), jnp.int32),
        pltpu.VMEM((2, 8, 128), jnp.int32),
        pltpu.VMEM((2, 128, D), jnp.float32),
        pltpu.VMEM_SHARED((NP, D), jnp.float32),
        pltpu.SemaphoreType.DMA,
        pltpu.SemaphoreType.DMA,
    ],
)(_sc_agg_body)


def _tc_y_body(x_ref, w_ref, h_ref, y_ref):
    xw = jnp.dot(x_ref[...], w_ref[...], preferred_element_type=jnp.float32)
    deg = 1.0 + jnp.sum(h_ref[0], axis=0)
    dinv = lax.rsqrt(deg)
    y_ref[...] = xw * dinv[:, None]


def _tc_y(x, w, hists):
    return pl.pallas_call(
        _tc_y_body,
        grid=(N // RB,),
        in_specs=[
            pl.BlockSpec((RB, D), lambda i: (i, 0)),
            pl.BlockSpec((D, D), lambda i: (0, 0)),
            pl.BlockSpec((1, NW, RB), lambda i: (i, 0, 0)),
        ],
        out_specs=pl.BlockSpec((RB, D), lambda i: (i, 0)),
        out_shape=jax.ShapeDtypeStruct((N, D), jnp.float32),
    )(x, w, hists)


def _tc_head_body(a0, a1, y, h, x_ref, bc, w1, b1, w2, b2, w3, b3, o_ref):
    deg = 1.0 + jnp.sum(h[0], axis=0)
    dinv = lax.rsqrt(deg)
    g = (a0[...] + a1[...] + y[...]) * dinv[:, None] + bc[...]
    g = jnp.maximum(g, 0.0) + x_ref[...]
    z = jnp.maximum(
        jnp.dot(g, w1[...], preferred_element_type=jnp.float32) + b1[...], 0.0)
    z = jnp.maximum(
        jnp.dot(z, w2[...], preferred_element_type=jnp.float32) + b2[...], 0.0)
    o_ref[...] = jnp.dot(z, w3[...], preferred_element_type=jnp.float32) + b3[...]


def _tc_head(acc0, acc1, y, hists, x, bc, w1, b1, w2, b2, w3, b3):
    full = lambda r, c: pl.BlockSpec((r, c), lambda i: (0, 0))
    return pl.pallas_call(
        _tc_head_body,
        grid=(N // RB,),
        in_specs=[
            pl.BlockSpec((RB, D), lambda i: (i, 0)),
            pl.BlockSpec((RB, D), lambda i: (i, 0)),
            pl.BlockSpec((RB, D), lambda i: (i, 0)),
            pl.BlockSpec((1, NW, RB), lambda i: (i, 0, 0)),
            pl.BlockSpec((RB, D), lambda i: (i, 0)),
            full(1, D), full(D, 32), full(1, 32),
            full(32, 32), full(1, 32), full(32, 2), full(1, 2),
        ],
        out_specs=pl.BlockSpec((RB, 2), lambda i: (i, 0)),
        out_shape=jax.ShapeDtypeStruct((N, 2), jnp.float32),
    )(acc0, acc1, y, hists, x, bc, w1, b1, w2, b2, w3, b3)


def kernel(x, edge_index, W_conv, b_conv, W1, b1, W2, b2, W3, b3):
    ei = edge_index.astype(jnp.int32)
    erows = ei.shape[1] // 128
    ei3 = jnp.pad(ei.reshape(2, erows, 128),
                  ((0, 0), (0, EROWS - erows), (0, 0)))

    hists = _sc_count(ei3)
    hs3 = hists[:, :N].reshape(NW, N // RB, RB).transpose(1, 0, 2)
    y = _tc_y(x, W_conv, hs3)
    acc0, acc1 = _sc_agg(y, ei3)

    return _tc_head(acc0, acc1, y, hs3, x, b_conv.reshape(1, D),
                    W1, b1.reshape(1, 32), W2, b2.reshape(1, 32),
                    W3, b3.reshape(1, 2))


# speedup vs baseline: 1.0252x; 1.0003x over previous
import functools

import jax
import jax.numpy as jnp
from jax import lax
from jax.experimental import pallas as pl
from jax.experimental.pallas import tpu as pltpu
from jax.experimental.pallas import tpu_sc as plsc

N = 10000
D = 128
NP = 10112
NC = 2
NS = 16
NW = NC * NS
ROWS_PT = 80
EROWS = NW * ROWS_PT
EROWS_REAL = 320000 // 128
RPS = NP // NS
RB = 2000

_mesh = plsc.VectorSubcoreMesh(
    core_axis_name="c", subcore_axis_name="s", num_cores=NC, num_subcores=NS)


def _wid():
    return lax.axis_index("s") * NC + lax.axis_index("c")


def _real_rows(wid):
    return jnp.clip(EROWS_REAL - wid * ROWS_PT, 0, ROWS_PT)


def _sc_count_body(ei_hbm, hist_hbm, didx_v, hist_v):
    wid = _wid()

    def zero(i, _):
        hist_v[pl.ds(i * 16, 16)] = jnp.zeros((16,), jnp.float32)
        return 0
    lax.fori_loop(0, NP // 16, zero, 0)

    pltpu.sync_copy(ei_hbm.at[1, pl.ds(wid * ROWS_PT, ROWS_PT)], didx_v)
    ones = jnp.ones((16,), jnp.float32)

    def row(r, _):
        for j in range(8):
            idx = didx_v[r, pl.ds(j * 16, 16)]
            plsc.addupdate_scatter(hist_v, [idx], ones)
        return 0
    lax.fori_loop(0, _real_rows(wid), row, 0)

    pltpu.sync_copy(hist_v, hist_hbm.at[wid])


_sc_count = functools.partial(
    pl.kernel,
    out_type=jax.ShapeDtypeStruct((NW, NP), jnp.float32),
    mesh=_mesh,
    compiler_params=pltpu.CompilerParams(needs_layout_passes=False),
    scratch_types=[
        pltpu.VMEM((ROWS_PT, 128), jnp.int32),
        pltpu.VMEM((NP,), jnp.float32),
    ],
)(_sc_count_body)


def _sc_agg_body(y_hbm, ei_hbm, acc0_hbm, acc1_hbm,
                 sidx_v, didx_v, rows_v, acc_sh, gsem):
    cid = lax.axis_index("c")
    sid = lax.axis_index("s")
    wid = _wid()

    base = sid * RPS

    def zrow(i, _):
        for j in range(8):
            rows_v[0, i, pl.ds(j * 16, 16)] = jnp.zeros((16,), jnp.float32)
        return 0
    lax.fori_loop(0, 128, zrow, 0)

    for k in range(4):
        pltpu.sync_copy(rows_v.at[0], acc_sh.at[pl.ds(base + k * 128, 128)])
    pltpu.sync_copy(rows_v.at[0, pl.ds(0, RPS - 512)],
                    acc_sh.at[pl.ds(base + 512, RPS - 512)])
    plsc.subcore_barrier()

    pltpu.sync_copy(ei_hbm.at[0, pl.ds(wid * ROWS_PT, ROWS_PT)], sidx_v)

    rw = _real_rows(wid)

    @pl.when(rw > 0)
    def _():
        pltpu.async_copy(y_hbm.at[sidx_v.at[0]], rows_v.at[0], gsem)

    def group(g, _):
        gn = jnp.minimum(8, rw - g * 8)
        pltpu.sync_copy(
            ei_hbm.at[1, pl.ds(wid * ROWS_PT + g * 8, 8)], didx_v)

        def chunk(j, _):
            c = g * 8 + j
            buf = lax.rem(c, 2)
            pltpu.make_async_copy(
                y_hbm.at[sidx_v.at[c]], rows_v.at[buf], gsem).wait()

            @pl.when(c + 1 < rw)
            def _():
                pltpu.async_copy(
                    y_hbm.at[sidx_v.at[c + 1]], rows_v.at[1 - buf], gsem)
            pltpu.sync_copy(rows_v.at[buf], acc_sh.at[didx_v.at[j]],
                            add=True)
            return 0
        lax.fori_loop(0, gn, chunk, 0)
        return 0
    lax.fori_loop(0, (rw + 7) // 8, group, 0)

    plsc.subcore_barrier()

    @pl.when(cid == 0)
    def _():
        pltpu.sync_copy(acc_sh.at[pl.ds(base, RPS)],
                        acc0_hbm.at[pl.ds(base, RPS)])

    @pl.when(cid == 1)
    def _():
        pltpu.sync_copy(acc_sh.at[pl.ds(base, RPS)],
                        acc1_hbm.at[pl.ds(base, RPS)])


_sc_agg = functools.partial(
    pl.kernel,
    out_type=[jax.ShapeDtypeStruct((NP, D), jnp.float32),
              jax.ShapeDtypeStruct((NP, D), jnp.float32)],
    mesh=_mesh,
    compiler_params=pltpu.CompilerParams(needs_layout_passes=False),
    scratch_types=[
        pltpu.VMEM((ROWS_PT, 128), jnp.int32),
        pltpu.VMEM((8, 128), jnp.int32),
        pltpu.VMEM((2, 128, D), jnp.float32),
        pltpu.VMEM_SHARED((NP, D), jnp.float32),
        pltpu.SemaphoreType.DMA,
    ],
)(_sc_agg_body)


def _tc_y_body(x_ref, w_ref, h_ref, y_ref):
    xw = jnp.dot(x_ref[...], w_ref[...], preferred_element_type=jnp.float32)
    deg = 1.0 + jnp.sum(h_ref[0], axis=0)
    dinv = lax.rsqrt(deg)
    y_ref[...] = xw * dinv[:, None]


def _tc_y(x, w, hists):
    return pl.pallas_call(
        _tc_y_body,
        grid=(N // RB,),
        in_specs=[
            pl.BlockSpec((RB, D), lambda i: (i, 0)),
            pl.BlockSpec((D, D), lambda i: (0, 0)),
            pl.BlockSpec((1, NW, RB), lambda i: (i, 0, 0)),
        ],
        out_specs=pl.BlockSpec((RB, D), lambda i: (i, 0)),
        out_shape=jax.ShapeDtypeStruct((N, D), jnp.float32),
    )(x, w, hists)


def _tc_head_body(a0, a1, y, h, x_ref, bc, w1, b1, w2, b2, w3, b3, o_ref):
    deg = 1.0 + jnp.sum(h[0], axis=0)
    dinv = lax.rsqrt(deg)
    g = (a0[...] + a1[...] + y[...]) * dinv[:, None] + bc[...]
    g = jnp.maximum(g, 0.0) + x_ref[...]
    z = jnp.maximum(
        jnp.dot(g, w1[...], preferred_element_type=jnp.float32) + b1[...], 0.0)
    z = jnp.maximum(
        jnp.dot(z, w2[...], preferred_element_type=jnp.float32) + b2[...], 0.0)
    o_ref[...] = jnp.dot(z, w3[...], preferred_element_type=jnp.float32) + b3[...]


def _tc_head(acc0, acc1, y, hists, x, bc, w1, b1, w2, b2, w3, b3):
    full = lambda r, c: pl.BlockSpec((r, c), lambda i: (0, 0))
    return pl.pallas_call(
        _tc_head_body,
        grid=(N // RB,),
        in_specs=[
            pl.BlockSpec((RB, D), lambda i: (i, 0)),
            pl.BlockSpec((RB, D), lambda i: (i, 0)),
            pl.BlockSpec((RB, D), lambda i: (i, 0)),
            pl.BlockSpec((1, NW, RB), lambda i: (i, 0, 0)),
            pl.BlockSpec((RB, D), lambda i: (i, 0)),
            full(1, D), full(D, 32), full(1, 32),
            full(32, 32), full(1, 32), full(32, 2), full(1, 2),
        ],
        out_specs=pl.BlockSpec((RB, 2), lambda i: (i, 0)),
        out_shape=jax.ShapeDtypeStruct((N, 2), jnp.float32),
    )(acc0, acc1, y, hists, x, bc, w1, b1, w2, b2, w3, b3)


def kernel(x, edge_index, W_conv, b_conv, W1, b1, W2, b2, W3, b3):
    ei = edge_index.astype(jnp.int32)
    erows = ei.shape[1] // 128
    ei3 = jnp.pad(ei.reshape(2, erows, 128),
                  ((0, 0), (0, EROWS - erows), (0, 0)))

    hists = _sc_count(ei3)
    hs3 = hists[:, :N].reshape(NW, N // RB, RB).transpose(1, 0, 2)
    y = _tc_y(x, W_conv, hs3)
    acc0, acc1 = _sc_agg(y, ei3)

    return _tc_head(acc0, acc1, y, hs3, x, b_conv.reshape(1, D),
                    W1, b1.reshape(1, 32), W2, b2.reshape(1, 32),
                    W3, b3.reshape(1, 2))
